# baseline (speedup 1.0000x reference)
import functools

import jax
import jax.numpy as jnp
from jax import lax
from jax.experimental import pallas as pl
from jax.experimental.pallas import tpu as pltpu
from jax.experimental.pallas import tpu_sc as plsc

N_NODES = 10000
N_EDGES = 320000
D = 128
C_PAD = 128

NC, NS = 2, 16
NW = NC * NS
CH = 128
N_PAD = 10240
NODE_CHUNKS = N_PAD // CH
CPT = 80
S = 16
NPASS = CPT // S
E_PAD = NW * CPT * CH
ROWS_PER_SUB = N_PAD // NS


def _sc_gnn(vocab1_hbm, vocab2_hbm, src2_hbm, dst2_hbm, emb_hbm,
            raw_hbm, aggp_hbm,
            srcA, dstA, vidA, srcB, dstB, vidB, idx0, mA, mB, agg_sh,
            semA, semB, semVA, semVB, semSA, semSB):
  c = lax.axis_index("c")
  s = lax.axis_index("s")
  w = s * NC + c

  with jax.named_scope("zero"):
    def _zrow(r, carry):
      for k in range(8):
        mA[r, pl.ds(16 * k, 16)] = jnp.zeros((16,), jnp.float32)
      return carry

    lax.fori_loop(0, CH, _zrow, 0)
    base_r = s * ROWS_PER_SUB
    for k in range(ROWS_PER_SUB // CH):
      pltpu.sync_copy(mA, agg_sh.at[pl.ds(base_r + k * CH, CH)])
    plsc.subcore_barrier()

  erow = w * CPT

  def _stage(prow, srcX, dstX, vidX, semX):
    pltpu.sync_copy(src2_hbm.at[pl.ds(prow, S)], srcX)
    pltpu.sync_copy(dst2_hbm.at[pl.ds(prow, S)], dstX)

    def _fire(j, carry):
      pltpu.async_copy(vocab1_hbm.at[srcX.at[j]], vidX.at[j], semX)
      return carry

    lax.fori_loop(0, S, _fire, 0)

  def _pipe(prow, dstX, vidX, semX):
    pltpu.make_async_copy(src2_hbm.at[pl.ds(prow, S)], vidX, semX).wait()
    pltpu.async_copy(emb_hbm.at[vidX.at[0]], mA, semA)

    def _step(jj, carry):
      j = 2 * jj

      @pl.when(jj > 0)
      def _():
        pltpu.make_async_copy(mB, agg_sh.at[dstX.at[j - 1]], semSB).wait()

      pltpu.async_copy(emb_hbm.at[vidX.at[j + 1]], mB, semB)
      pltpu.make_async_copy(emb_hbm.at[vidX.at[j]], mA, semA).wait()
      pltpu.async_copy(mA, agg_sh.at[dstX.at[j]], semSA, add=True)

      @pl.when(jj < S // 2 - 1)
      def _():
        pltpu.make_async_copy(mA, agg_sh.at[dstX.at[j]], semSA).wait()
        pltpu.async_copy(emb_hbm.at[vidX.at[j + 2]], mA, semA)

      pltpu.make_async_copy(emb_hbm.at[vidX.at[j + 1]], mB, semB).wait()
      pltpu.async_copy(mB, agg_sh.at[dstX.at[j + 1]], semSB, add=True)
      return carry

    lax.fori_loop(0, S // 2, _step, 0)
    pltpu.make_async_copy(mA, agg_sh.at[dstX.at[S - 2]], semSA).wait()
    pltpu.make_async_copy(mB, agg_sh.at[dstX.at[S - 1]], semSB).wait()

  with jax.named_scope("stage0"):
    _stage(erow, srcA, dstA, vidA, semVA)

  with jax.named_scope("phase0"):
    for jj in range(3):
      q = w + NW * jj

      @pl.when(q < NODE_CHUNKS)
      def _():
        pltpu.sync_copy(vocab2_hbm.at[q], idx0)
        pltpu.async_copy(emb_hbm.at[idx0], mA, semA).wait()
        pltpu.sync_copy(mA, raw_hbm.at[pl.ds(q * CH, CH)])

  sets = [(srcA, dstA, vidA, semVA), (srcB, dstB, vidB, semVB)]
  for p in range(NPASS):
    _, dstX, vidX, semX = sets[p % 2]
    if p + 1 < NPASS:
      with jax.named_scope("stage"):
        _stage(erow + (p + 1) * S, *sets[(p + 1) % 2])
    with jax.named_scope("pipe"):
      _pipe(erow + p * S, dstX, vidX, semX)

  with jax.named_scope("writeout"):
    plsc.subcore_barrier()
    for k in range(ROWS_PER_SUB // CH):
      pltpu.sync_copy(agg_sh.at[pl.ds(base_r + k * CH, CH)], mA)
      pltpu.sync_copy(mA, aggp_hbm.at[c, pl.ds(base_r + k * CH, CH)])


_sc_call = functools.partial(
    pl.kernel,
    out_type=(
        jax.ShapeDtypeStruct((N_PAD, D), jnp.float32),
        jax.ShapeDtypeStruct((NC, N_PAD, D), jnp.float32),
    ),
    mesh=plsc.VectorSubcoreMesh(core_axis_name="c", subcore_axis_name="s"),
    scratch_types=[
        pltpu.VMEM((S, CH), jnp.int32),
        pltpu.VMEM((S, CH), jnp.int32),
        pltpu.VMEM((S, CH), jnp.int32),
        pltpu.VMEM((S, CH), jnp.int32),
        pltpu.VMEM((S, CH), jnp.int32),
        pltpu.VMEM((S, CH), jnp.int32),
        pltpu.VMEM((CH,), jnp.int32),
        pltpu.VMEM((CH, D), jnp.float32),
        pltpu.VMEM((CH, D), jnp.float32),
        pltpu.VMEM_SHARED((N_PAD, D), jnp.float32),
        pltpu.SemaphoreType.DMA,
        pltpu.SemaphoreType.DMA,
        pltpu.SemaphoreType.DMA,
        pltpu.SemaphoreType.DMA,
        pltpu.SemaphoreType.DMA,
        pltpu.SemaphoreType.DMA,
    ],
)(_sc_gnn)


def _tc_dense(raw_ref, agg0_ref, agg1_ref, lab_ref, ws_ref, wn_ref, bg_ref,
              wo1_ref, wo2_ref, bo_ref, logits_ref, loss_ref):
  raw = raw_ref[...]
  agg = agg0_ref[...] + agg1_ref[...]
  ro = jnp.maximum(
      jnp.dot(raw, ws_ref[...], preferred_element_type=jnp.float32)
      + jnp.dot(agg, wn_ref[...], preferred_element_type=jnp.float32)
      + bg_ref[...], 0.0)
  logits = (jnp.dot(raw, wo1_ref[...], preferred_element_type=jnp.float32)
            + jnp.dot(ro, wo2_ref[...], preferred_element_type=jnp.float32)
            + bo_ref[...])
  logits_ref[...] = logits
  m = jnp.max(logits, axis=1, keepdims=True)
  lse = m + jnp.log(jnp.sum(jnp.exp(logits - m), axis=1, keepdims=True))
  cls = lax.broadcasted_iota(jnp.int32, (N_PAD, C_PAD), 1).astype(jnp.float32)
  onehot = cls == lab_ref[...]
  label_logit = jnp.sum(jnp.where(onehot, logits, 0.0), axis=1, keepdims=True)
  rowid = lax.broadcasted_iota(jnp.int32, (N_PAD, 1), 0)
  nll = jnp.where(rowid < N_NODES, lse - label_logit, 0.0)
  loss_ref[...] = jnp.reshape(jnp.sum(nll), (1, 1))


def kernel(vocab_ids, labels, edge_lists, emb_table, W_self, W_nbr, b_gnn,
           W_out, b_out):
  n_classes = W_out.shape[1]
  vocab_pad = jnp.zeros((N_PAD,), jnp.int32).at[:N_NODES].set(
      vocab_ids.astype(jnp.int32))
  pad_src = jnp.arange(E_PAD, dtype=jnp.int32) % N_NODES
  src_pad = pad_src.at[:N_EDGES].set(edge_lists[0].astype(jnp.int32))
  pad_dst = (jnp.arange(E_PAD, dtype=jnp.int32) % (N_PAD - N_NODES)) + N_NODES
  dst_pad = pad_dst.at[:N_EDGES].set(edge_lists[1].astype(jnp.int32))
  src2 = src_pad.reshape(NW * CPT, CH)
  dst2 = dst_pad.reshape(NW * CPT, CH)
  vocab2 = vocab_pad.reshape(NODE_CHUNKS, CH)

  raw_pad, aggp = _sc_call(vocab_pad, vocab2, src2, dst2, emb_table)

  lab_pad = jnp.zeros((N_PAD, 1), jnp.float32).at[:N_NODES, 0].set(
      labels.astype(jnp.float32))
  wo1 = jnp.zeros((D, C_PAD), jnp.float32).at[:, :n_classes].set(W_out[:D])
  wo2 = jnp.zeros((D, C_PAD), jnp.float32).at[:, :n_classes].set(W_out[D:])
  bo2 = jnp.full((1, C_PAD), -1e30, jnp.float32).at[0, :n_classes].set(b_out)
  bg2 = b_gnn.reshape(1, D)

  logits_pad, loss_sum = pl.pallas_call(
      _tc_dense,
      out_shape=(
          jax.ShapeDtypeStruct((N_PAD, C_PAD), jnp.float32),
          jax.ShapeDtypeStruct((1, 1), jnp.float32),
      ),
  )(raw_pad, aggp[0], aggp[1], lab_pad, W_self, W_nbr, bg2, wo1, wo2, bo2)

  logits = logits_pad[:N_NODES, :n_classes]
  loss = loss_sum[0, 0] / N_NODES
  return (logits, loss)


# speedup vs baseline: 1.0099x; 1.0099x over previous
import functools

import jax
import jax.numpy as jnp
from jax import lax
from jax.experimental import pallas as pl
from jax.experimental.pallas import tpu as pltpu
from jax.experimental.pallas import tpu_sc as plsc

N_NODES = 10000
N_EDGES = 320000
D = 128
C_PAD = 128

NC, NS = 2, 16
NW = NC * NS
CH = 128
N_PAD = 10240
NODE_CHUNKS = N_PAD // CH
CPT = 80
S = 16
NPASS = CPT // S
E_PAD = NW * CPT * CH
ROWS_PER_SUB = N_PAD // NS


def _sc_gnn(vocab1_hbm, vocab2_hbm, src2_hbm, dst2_hbm, emb_hbm,
            raw_hbm, aggp_hbm,
            srcA, dstA, vidA, srcB, dstB, vidB, idx0, mA, mB, agg_sh,
            semA, semB, semVA, semVB, semSA, semSB):
  c = lax.axis_index("c")
  s = lax.axis_index("s")
  w = s * NC + c

  with jax.named_scope("zero"):
    def _zrow(r, carry):
      for k in range(8):
        mA[r, pl.ds(16 * k, 16)] = jnp.zeros((16,), jnp.float32)
      return carry

    lax.fori_loop(0, CH, _zrow, 0)
    base_r = s * ROWS_PER_SUB
    for k in range(ROWS_PER_SUB // CH):
      pltpu.sync_copy(mA, agg_sh.at[pl.ds(base_r + k * CH, CH)])
    plsc.subcore_barrier()

  erow = w * CPT

  def _stage(prow, srcX, dstX, vidX, semX):
    pltpu.sync_copy(src2_hbm.at[pl.ds(prow, S)], srcX)
    pltpu.sync_copy(dst2_hbm.at[pl.ds(prow, S)], dstX)

    def _fire(j, carry):
      pltpu.async_copy(vocab1_hbm.at[srcX.at[j]], vidX.at[j], semX)
      return carry

    lax.fori_loop(0, S, _fire, 0)

  def _pipe(prow, dstX, vidX, semX):
    pltpu.make_async_copy(src2_hbm.at[pl.ds(prow, S)], vidX, semX).wait()
    pltpu.async_copy(emb_hbm.at[vidX.at[0]], mA, semA)

    def _step(jj, carry):
      j = 2 * jj

      @pl.when(jj > 0)
      def _():
        pltpu.make_async_copy(mB, agg_sh.at[dstX.at[j - 1]], semSB).wait()

      pltpu.async_copy(emb_hbm.at[vidX.at[j + 1]], mB, semB)
      pltpu.make_async_copy(emb_hbm.at[vidX.at[j]], mA, semA).wait()
      pltpu.async_copy(mA, agg_sh.at[dstX.at[j]], semSA, add=True)

      @pl.when(jj < S // 2 - 1)
      def _():
        pltpu.make_async_copy(mA, agg_sh.at[dstX.at[j]], semSA).wait()
        pltpu.async_copy(emb_hbm.at[vidX.at[j + 2]], mA, semA)

      pltpu.make_async_copy(emb_hbm.at[vidX.at[j + 1]], mB, semB).wait()
      pltpu.async_copy(mB, agg_sh.at[dstX.at[j + 1]], semSB, add=True)
      return carry

    lax.fori_loop(0, S // 2, _step, 0)
    pltpu.make_async_copy(mA, agg_sh.at[dstX.at[S - 2]], semSA).wait()
    pltpu.make_async_copy(mB, agg_sh.at[dstX.at[S - 1]], semSB).wait()

  with jax.named_scope("stage0"):
    _stage(erow, srcA, dstA, vidA, semVA)

  with jax.named_scope("phase0"):
    for jj in range(3):
      q = w + NW * jj

      @pl.when(q < NODE_CHUNKS)
      def _():
        pltpu.sync_copy(vocab2_hbm.at[q], idx0)
        pltpu.async_copy(emb_hbm.at[idx0], mA, semA).wait()
        pltpu.sync_copy(mA, raw_hbm.at[pl.ds(q * CH, CH)])

  sets = [(srcA, dstA, vidA, semVA), (srcB, dstB, vidB, semVB)]
  for p in range(NPASS):
    _, dstX, vidX, semX = sets[p % 2]
    if p + 1 < NPASS:
      with jax.named_scope("stage"):
        _stage(erow + (p + 1) * S, *sets[(p + 1) % 2])
    with jax.named_scope("pipe"):
      _pipe(erow + p * S, dstX, vidX, semX)

  with jax.named_scope("writeout"):
    plsc.subcore_barrier()
    for k in range(ROWS_PER_SUB // CH):
      pltpu.sync_copy(agg_sh.at[pl.ds(base_r + k * CH, CH)], mA)
      pltpu.sync_copy(mA, aggp_hbm.at[c, pl.ds(base_r + k * CH, CH)])


_sc_call = functools.partial(
    pl.kernel,
    out_type=(
        jax.ShapeDtypeStruct((N_PAD, D), jnp.float32),
        jax.ShapeDtypeStruct((NC, N_PAD, D), jnp.float32),
    ),
    mesh=plsc.VectorSubcoreMesh(core_axis_name="c", subcore_axis_name="s"),
    scratch_types=[
        pltpu.VMEM((S, CH), jnp.int32),
        pltpu.VMEM((S, CH), jnp.int32),
        pltpu.VMEM((S, CH), jnp.int32),
        pltpu.VMEM((S, CH), jnp.int32),
        pltpu.VMEM((S, CH), jnp.int32),
        pltpu.VMEM((S, CH), jnp.int32),
        pltpu.VMEM((CH,), jnp.int32),
        pltpu.VMEM((CH, D), jnp.float32),
        pltpu.VMEM((CH, D), jnp.float32),
        pltpu.VMEM_SHARED((N_PAD, D), jnp.float32),
        pltpu.SemaphoreType.DMA,
        pltpu.SemaphoreType.DMA,
        pltpu.SemaphoreType.DMA,
        pltpu.SemaphoreType.DMA,
        pltpu.SemaphoreType.DMA,
        pltpu.SemaphoreType.DMA,
    ],
)(_sc_gnn)


def _tc_dense(raw_ref, agg0_ref, agg1_ref, lab_ref, ws_ref, wn_ref, bg_ref,
              wo1_ref, wo2_ref, bo_ref, logits_ref, loss_ref):
  raw = raw_ref[...]
  agg = agg0_ref[...] + agg1_ref[...]
  ro = jnp.maximum(
      jnp.dot(raw, ws_ref[...], preferred_element_type=jnp.float32)
      + jnp.dot(agg, wn_ref[...], preferred_element_type=jnp.float32)
      + bg_ref[...], 0.0)
  logits = (jnp.dot(raw, wo1_ref[...], preferred_element_type=jnp.float32)
            + jnp.dot(ro, wo2_ref[...], preferred_element_type=jnp.float32)
            + bo_ref[...])
  logits_ref[...] = logits[:, :logits_ref.shape[1]]
  m = jnp.max(logits, axis=1, keepdims=True)
  lse = m + jnp.log(jnp.sum(jnp.exp(logits - m), axis=1, keepdims=True))
  cls = lax.broadcasted_iota(jnp.int32, (N_PAD, C_PAD), 1).astype(jnp.float32)
  onehot = cls == lab_ref[...]
  label_logit = jnp.sum(jnp.where(onehot, logits, 0.0), axis=1, keepdims=True)
  rowid = lax.broadcasted_iota(jnp.int32, (N_PAD, 1), 0)
  nll = jnp.where(rowid < N_NODES, lse - label_logit, 0.0)
  loss_ref[...] = jnp.reshape(jnp.sum(nll) * (1.0 / N_NODES), (1, 1))


def kernel(vocab_ids, labels, edge_lists, emb_table, W_self, W_nbr, b_gnn,
           W_out, b_out):
  n_classes = W_out.shape[1]
  vocab_pad = jnp.zeros((N_PAD,), jnp.int32).at[:N_NODES].set(
      vocab_ids.astype(jnp.int32))
  pad_src = jnp.arange(E_PAD, dtype=jnp.int32) % N_NODES
  src_pad = pad_src.at[:N_EDGES].set(edge_lists[0].astype(jnp.int32))
  pad_dst = (jnp.arange(E_PAD, dtype=jnp.int32) % (N_PAD - N_NODES)) + N_NODES
  dst_pad = pad_dst.at[:N_EDGES].set(edge_lists[1].astype(jnp.int32))
  src2 = src_pad.reshape(NW * CPT, CH)
  dst2 = dst_pad.reshape(NW * CPT, CH)
  vocab2 = vocab_pad.reshape(NODE_CHUNKS, CH)

  raw_pad, aggp = _sc_call(vocab_pad, vocab2, src2, dst2, emb_table)

  lab_pad = jnp.zeros((N_PAD, 1), jnp.float32).at[:N_NODES, 0].set(
      labels.astype(jnp.float32))
  wo1 = jnp.zeros((D, C_PAD), jnp.float32).at[:, :n_classes].set(W_out[:D])
  wo2 = jnp.zeros((D, C_PAD), jnp.float32).at[:, :n_classes].set(W_out[D:])
  bo2 = jnp.full((1, C_PAD), -1e30, jnp.float32).at[0, :n_classes].set(b_out)
  bg2 = b_gnn.reshape(1, D)

  logits_pad, loss_out = pl.pallas_call(
      _tc_dense,
      out_shape=(
          jax.ShapeDtypeStruct((N_PAD, n_classes), jnp.float32),
          jax.ShapeDtypeStruct((1, 1), jnp.float32),
      ),
  )(raw_pad, aggp[0], aggp[1], lab_pad, W_self, W_nbr, bg2, wo1, wo2, bo2)

  logits = logits_pad[:N_NODES]
  loss = loss_out[0, 0]
  return (logits, loss)
